# baseline (speedup 1.0000x reference)
import jax
import jax.numpy as jnp
from jax.experimental import pallas as pl
from jax.experimental.pallas import tpu as pltpu

N = 10000
NFEAT = 128
NHID = 16
NCLASS = 7
BR = 200
NB = N // BR
K = 4


def _copy_block(adj_ref, buf_ref, sem, g, slot):
    p = g // NB
    r = (g - p * NB) * BR
    pltpu.make_async_copy(
        adj_ref.at[p, pl.ds(r, BR), :], buf_ref.at[slot], sem.at[slot]
    ).start()


def _body(adj_ref, x_ref, w1_ref, b1_ref, w2_ref, b2_ref, wl_ref, bl_ref,
          out_ref, buf_ref, s1_scr, s2_scr, sem):
    for g in range(K):
        _copy_block(adj_ref, buf_ref, sem, g, g)

    s1_scr[...] = jnp.dot(x_ref[...], w1_ref[...],
                          preferred_element_type=jnp.float32)

    def step(g, carry):
        slot = jax.lax.rem(g, K)
        i = jax.lax.rem(g, NB)
        pltpu.make_async_copy(
            adj_ref.at[0, pl.ds(0, BR), :], buf_ref.at[slot], sem.at[slot]
        ).wait()
        blk = buf_ref[slot]

        @pl.when(g < NB)
        def _():
            h = jnp.dot(blk, s1_scr[...], preferred_element_type=jnp.float32)
            h = jnp.maximum(h + b1_ref[...], 0.0)
            s2_scr[pl.ds(i * BR, BR), :] = jnp.dot(
                h, w2_ref[...], preferred_element_type=jnp.float32)

        @pl.when(g >= NB)
        def _():
            h2 = jnp.dot(blk, s2_scr[...],
                         preferred_element_type=jnp.float32) + b2_ref[...]
            o = jnp.dot(h2, wl_ref[...],
                        preferred_element_type=jnp.float32) + bl_ref[...]
            m = jnp.max(o, axis=-1, keepdims=True)
            e = o - m
            out_ref[pl.ds(i * BR, BR), :] = e - jnp.log(
                jnp.sum(jnp.exp(e), axis=-1, keepdims=True))

        @pl.when(g + K < 2 * NB)
        def _():
            _copy_block(adj_ref, buf_ref, sem, g + K, slot)

        return carry

    jax.lax.fori_loop(0, 2 * NB, step, 0)


def kernel(x, adj, W1, b1, W2, b2, WL, bL):
    b1r = b1.reshape(1, NHID)
    b2r = b2.reshape(1, NCLASS)
    bLr = bL.reshape(1, NCLASS)
    vmem = pl.BlockSpec(memory_space=pltpu.MemorySpace.VMEM)
    return pl.pallas_call(
        _body,
        in_specs=[
            pl.BlockSpec(memory_space=pltpu.MemorySpace.HBM),
            vmem, vmem, vmem, vmem, vmem, vmem, vmem,
        ],
        out_specs=pl.BlockSpec(memory_space=pltpu.MemorySpace.VMEM),
        out_shape=jax.ShapeDtypeStruct((N, NCLASS), jnp.float32),
        scratch_shapes=[
            pltpu.VMEM((K, BR, N), jnp.float32),
            pltpu.VMEM((N, NHID), jnp.float32),
            pltpu.VMEM((N, NCLASS), jnp.float32),
            pltpu.SemaphoreType.DMA((K,)),
        ],
    )(adj, x, W1, b1r, W2, b2r, WL, bLr)


# speedup vs baseline: 1.0008x; 1.0008x over previous
import jax
import jax.numpy as jnp
from jax.experimental import pallas as pl
from jax.experimental.pallas import tpu as pltpu

N = 10000
NFEAT = 128
NHID = 16
NCLASS = 7
BR = 400
NB = N // BR


def _body(adj_ref, x_ref, w1_ref, b1_ref, w2l_ref, c_ref,
          out_ref, s1_scr, h_scr, t_scr):
    g = pl.program_id(0)
    i = jax.lax.rem(g, NB)

    @pl.when(g == 0)
    def _():
        s1_scr[...] = jnp.dot(x_ref[...], w1_ref[...],
                              preferred_element_type=jnp.float32)

    @pl.when(g < NB)
    def _():
        h = jnp.dot(adj_ref[0], s1_scr[...],
                    preferred_element_type=jnp.float32)
        h_scr[pl.ds(i * BR, BR), :] = jnp.maximum(h + b1_ref[...], 0.0)

    @pl.when(g == NB)
    def _():
        t_scr[...] = jnp.dot(h_scr[...], w2l_ref[...],
                             preferred_element_type=jnp.float32)

    @pl.when(g >= NB)
    def _():
        o = jnp.dot(adj_ref[0], t_scr[...],
                    preferred_element_type=jnp.float32) + c_ref[...]
        m = jnp.max(o, axis=-1, keepdims=True)
        e = o - m
        out_ref[pl.ds(i * BR, BR), :] = e - jnp.log(
            jnp.sum(jnp.exp(e), axis=-1, keepdims=True))


def kernel(x, adj, W1, b1, W2, b2, WL, bL):
    b1r = b1.reshape(1, NHID)
    W2L = W2 @ WL
    c = (b2 @ WL + bL).reshape(1, NCLASS)
    cidx = lambda g: (0, 0)
    return pl.pallas_call(
        _body,
        grid=(2 * NB,),
        in_specs=[
            pl.BlockSpec((1, BR, N), lambda g: (g // NB, g % NB, 0)),
            pl.BlockSpec((N, NFEAT), cidx),
            pl.BlockSpec((NFEAT, NHID), cidx),
            pl.BlockSpec((1, NHID), cidx),
            pl.BlockSpec((NHID, NCLASS), cidx),
            pl.BlockSpec((1, NCLASS), cidx),
        ],
        out_specs=pl.BlockSpec((N, NCLASS), cidx),
        out_shape=jax.ShapeDtypeStruct((N, NCLASS), jnp.float32),
        scratch_shapes=[
            pltpu.VMEM((N, NHID), jnp.float32),
            pltpu.VMEM((N, NHID), jnp.float32),
            pltpu.VMEM((N, NCLASS), jnp.float32),
        ],
    )(adj, x, W1, b1r, W2L, c)


# speedup vs baseline: 1.0078x; 1.0070x over previous
import jax
import jax.numpy as jnp
from jax.experimental import pallas as pl
from jax.experimental.pallas import tpu as pltpu

N = 10000
NFEAT = 128
NHID = 16
NCLASS = 7
BR = 400
NB = N // BR


def _body(adj_ref, x_ref, w1_ref, b1_ref, w2_ref, b2_ref, wl_ref, bl_ref,
          out_ref, s1_scr, s2_scr):
    p = pl.program_id(0)
    i = pl.program_id(1)

    @pl.when((p == 0) & (i == 0))
    def _():
        s1_scr[...] = jnp.dot(x_ref[...], w1_ref[...],
                              preferred_element_type=jnp.float32)

    @pl.when(p == 0)
    def _():
        h = jnp.dot(adj_ref[0], s1_scr[...],
                    preferred_element_type=jnp.float32)
        h = jnp.maximum(h + b1_ref[...], 0.0)
        s2_scr[pl.ds(i * BR, BR), :] = jnp.dot(
            h, w2_ref[...], preferred_element_type=jnp.float32)

    @pl.when(p == 1)
    def _():
        h2 = jnp.dot(adj_ref[0], s2_scr[...],
                     preferred_element_type=jnp.float32) + b2_ref[...]
        o = jnp.dot(h2, wl_ref[...],
                    preferred_element_type=jnp.float32) + bl_ref[...]
        m = jnp.max(o, axis=-1, keepdims=True)
        e = o - m
        out_ref[pl.ds(i * BR, BR), :] = e - jnp.log(
            jnp.sum(jnp.exp(e), axis=-1, keepdims=True))


def kernel(x, adj, W1, b1, W2, b2, WL, bL):
    b1r = b1.reshape(1, NHID)
    b2r = b2.reshape(1, NCLASS)
    bLr = bL.reshape(1, NCLASS)
    c = lambda p, i: (0, 0)
    return pl.pallas_call(
        _body,
        grid=(2, NB),
        in_specs=[
            pl.BlockSpec((1, BR, N), lambda p, i: (p, i, 0)),
            pl.BlockSpec((N, NFEAT), c),
            pl.BlockSpec((NFEAT, NHID), c),
            pl.BlockSpec((1, NHID), c),
            pl.BlockSpec((NHID, NCLASS), c),
            pl.BlockSpec((1, NCLASS), c),
            pl.BlockSpec((NCLASS, NCLASS), c),
            pl.BlockSpec((1, NCLASS), c),
        ],
        out_specs=pl.BlockSpec((N, NCLASS), c),
        out_shape=jax.ShapeDtypeStruct((N, NCLASS), jnp.float32),
        scratch_shapes=[
            pltpu.VMEM((N, NHID), jnp.float32),
            pltpu.VMEM((N, NCLASS), jnp.float32),
        ],
    )(adj, x, W1, b1r, W2, b2r, WL, bLr)
